# baseline (speedup 1.0000x reference)
import functools

import numpy as np

import jax
import jax.numpy as jnp
from jax.experimental import pallas as pl
from jax.experimental.pallas import tpu as pltpu
from jax.sharding import Mesh, PartitionSpec as P

_LANE = 128


def _round_up(v, m):
    return (v + m - 1) // m * m


def _bf16(a):
    return a.astype(jnp.bfloat16)


def _fs(shape):
    zeros = (0,) * len(shape)
    return pl.BlockSpec(shape, lambda *_, _z=zeros: _z)


def _call(body, out_shape, operands):
    return pl.pallas_call(
        body,
        out_shape=jax.ShapeDtypeStruct(out_shape, jnp.float32),
        grid=(1,),
        in_specs=[_fs(o.shape) for o in operands],
        out_specs=_fs(out_shape),
        compiler_params=pltpu.CompilerParams(
            dimension_semantics=("arbitrary",)),
    )(*operands)


def _embed_kernel(x_ref, na_ref, ew_ref, eb_ref, h_ref):
    xb = _bf16(x_ref[...] * na_ref[...])
    h_ref[...] = (
        jnp.dot(xb, ew_ref[...], preferred_element_type=jnp.float32)
        + eb_ref[...]
    )


def _layer_update(h16_ref, hloc_ref, adj_ref, deg_ref, ce_ref,
                  wi_ref, wj_ref, we_ref):
    hloc = hloc_ref[...]
    ah = jnp.dot(adj_ref[...], h16_ref[...],
                 preferred_element_type=jnp.float32)
    agg = (
        deg_ref[...] * jnp.dot(_bf16(hloc), wi_ref[...],
                               preferred_element_type=jnp.float32)
        + jnp.dot(_bf16(ah), wj_ref[...], preferred_element_type=jnp.float32)
        + jnp.dot(ce_ref[...], we_ref[...], preferred_element_type=jnp.float32)
    )
    return hloc + agg * jax.nn.sigmoid(agg)


def _layer_kernel(h16_ref, hloc_ref, adj_ref, deg_ref, ce_ref,
                  wi_ref, wj_ref, we_ref, o_ref):
    o_ref[...] = _layer_update(h16_ref, hloc_ref, adj_ref, deg_ref, ce_ref,
                               wi_ref, wj_ref, we_ref)


def _last_kernel(h16_ref, hloc_ref, adj_ref, deg_ref, ce_ref,
                 wi_ref, wj_ref, we_ref,
                 na_ref, w1_ref, b1_ref, w2_ref, b2_ref, o_ref):
    h_new = _layer_update(h16_ref, hloc_ref, adj_ref, deg_ref, ce_ref,
                          wi_ref, wj_ref, we_ref)
    na = na_ref[...]
    t = (jnp.dot(_bf16(h_new * na), w1_ref[...],
                 preferred_element_type=jnp.float32) + b1_ref[...])
    t = t * jax.nn.sigmoid(t)
    o_ref[...] = (
        jnp.dot(_bf16(t * na), w2_ref[...],
                preferred_element_type=jnp.float32) + b2_ref[...]
    )


def _forward(ndev, x, na, adj16, deg, ce16, ew, eb, lwi, lwj, lwe,
             w1, b1, w2, b2):
    h = _call(_embed_kernel, (x.shape[0], ew.shape[1]), (x, na, ew, eb))
    for l in range(6):
        h16 = _bf16(h)
        if ndev > 1:
            h16 = jax.lax.all_gather(h16, "c", axis=0, tiled=True)
        ops = (h16, h, adj16, deg, ce16, lwi[l], lwj[l], lwe[l])
        if l < 5:
            h = _call(_layer_kernel, h.shape, ops)
        else:
            out = _call(_last_kernel, (h.shape[0], w2.shape[1]),
                        ops + (na, w1, b1, w2, b2))
    return out


@jax.jit
def kernel(x, edge_index, amf, node_attr, edge_attr, embed_w, embed_b,
           out1_w, out1_b, out2_w, out2_b, layer0_w, layer0_b, layer1_w,
           layer1_b, layer2_w, layer2_b, layer3_w, layer3_b, layer4_w,
           layer4_b, layer5_w, layer5_b):
    n, in_dim = x.shape
    hidden = embed_w.shape[1]
    a_dim = amf.shape[1]
    ae_p = _round_up(a_dim + 1, _LANE)

    src, dst = edge_index[0], edge_index[1]
    ea = edge_attr[:, 0]
    adj = jnp.zeros((n, n), jnp.float32).at[dst, src].add(ea)
    deg = jnp.zeros((n, 1), jnp.float32).at[dst, 0].add(ea)
    c_amf = jnp.zeros((n, a_dim), jnp.float32).at[dst].add(edge_attr * amf)
    cnt = jnp.zeros((n, 1), jnp.float32).at[dst, 0].add(
        jnp.ones(ea.shape, jnp.float32))
    c_ext = jnp.pad(jnp.concatenate([c_amf, cnt], axis=1),
                    ((0, 0), (0, ae_p - (a_dim + 1))))
    adj16 = _bf16(adj)
    ce16 = _bf16(c_ext)

    lwi, lwj, lwe = [], [], []
    for w, b in ((layer0_w, layer0_b), (layer1_w, layer1_b),
                 (layer2_w, layer2_b), (layer3_w, layer3_b),
                 (layer4_w, layer4_b), (layer5_w, layer5_b)):
        lwi.append(_bf16(w[:hidden]))
        lwj.append(_bf16(w[hidden:2 * hidden]))
        we = jnp.concatenate([w[2 * hidden:2 * hidden + a_dim], b], axis=0)
        lwe.append(_bf16(jnp.pad(we, ((0, ae_p - (a_dim + 1)), (0, 0)))))
    lwi, lwj, lwe = jnp.stack(lwi), jnp.stack(lwj), jnp.stack(lwe)

    devs = jax.devices()
    ndev = 2 if (len(devs) >= 2 and n % 2 == 0) else 1
    mesh = Mesh(np.array(devs[:ndev]), ("c",))
    row = P("c", None)
    rep2 = P(None, None)
    rep3 = P(None, None, None)

    out = jax.shard_map(
        functools.partial(_forward, ndev),
        mesh=mesh,
        in_specs=(row, row, row, row, row, rep2, rep2, rep3, rep3, rep3,
                  rep2, rep2, rep2, rep2),
        out_specs=row,
        check_vma=False,
    )(x, node_attr, adj16, deg, ce16, _bf16(embed_w), embed_b,
      lwi, lwj, lwe, _bf16(out1_w), out1_b, _bf16(out2_w), out2_b)

    return out


# speedup vs baseline: 1.9431x; 1.9431x over previous
import functools

import numpy as np

import jax
import jax.numpy as jnp
from jax.experimental import pallas as pl
from jax.experimental.pallas import tpu as pltpu
from jax.sharding import Mesh, PartitionSpec as P

_LANE = 128
_ECHUNK = 2048


def _bf16(a):
    return a.astype(jnp.bfloat16)


def _fs(shape):
    zeros = (0,) * len(shape)
    return pl.BlockSpec(shape, lambda *_, _z=zeros: _z)


def _call(body, out_shape, operands):
    return pl.pallas_call(
        body,
        out_shape=jax.ShapeDtypeStruct(out_shape, jnp.float32),
        grid=(1,),
        in_specs=[_fs(o.shape) for o in operands],
        out_specs=_fs(out_shape),
        compiler_params=pltpu.CompilerParams(
            dimension_semantics=("arbitrary",)),
    )(*operands)


def _build_kernel(dst_ref, src_ref, ea_ref, v_ref, adj_ref, small_ref,
                  acc_ref, sacc_ref, *, blk, n, nchunks):
    k = pl.program_id(0)

    @pl.when(k == 0)
    def _():
        acc_ref[...] = jnp.zeros_like(acc_ref)
        sacc_ref[...] = jnp.zeros_like(sacc_ref)

    ec = dst_ref.shape[2]
    dstv = dst_ref[0]
    srcv = src_ref[0]
    eav = _bf16(ea_ref[0])
    ut = _bf16(
        jax.lax.broadcasted_iota(jnp.int32, (blk, ec), 0) == dstv)
    sp = _bf16(
        jax.lax.broadcasted_iota(jnp.int32, (ec, n), 1) == srcv) * eav
    acc_ref[...] += jnp.dot(ut, sp, preferred_element_type=jnp.float32)
    sacc_ref[...] += jnp.dot(ut, v_ref[0],
                             preferred_element_type=jnp.float32)

    @pl.when(k == nchunks - 1)
    def _():
        adj_ref[...] = _bf16(acc_ref[...])
        small_ref[...] = sacc_ref[...]


def _embed_kernel(x_ref, na_ref, ew_ref, eb_ref, h_ref):
    xb = _bf16(x_ref[...] * na_ref[...])
    h_ref[...] = (
        jnp.dot(xb, ew_ref[...], preferred_element_type=jnp.float32)
        + eb_ref[...]
    )


def _layer_update(h16_ref, hloc_ref, adj_ref, deg_ref, ce_ref,
                  wi_ref, wj_ref, we_ref):
    hloc = hloc_ref[...]
    ah = jnp.dot(adj_ref[...], h16_ref[...],
                 preferred_element_type=jnp.float32)
    agg = (
        deg_ref[...] * jnp.dot(_bf16(hloc), wi_ref[...],
                               preferred_element_type=jnp.float32)
        + jnp.dot(_bf16(ah), wj_ref[...], preferred_element_type=jnp.float32)
        + jnp.dot(ce_ref[...], we_ref[...], preferred_element_type=jnp.float32)
    )
    return hloc + agg * jax.nn.sigmoid(agg)


def _layer_kernel(h16_ref, hloc_ref, adj_ref, deg_ref, ce_ref,
                  wi_ref, wj_ref, we_ref, o_ref):
    o_ref[...] = _layer_update(h16_ref, hloc_ref, adj_ref, deg_ref, ce_ref,
                               wi_ref, wj_ref, we_ref)


def _last_kernel(h16_ref, hloc_ref, adj_ref, deg_ref, ce_ref,
                 wi_ref, wj_ref, we_ref,
                 na_ref, w1_ref, b1_ref, w2_ref, b2_ref, o_ref):
    h_new = _layer_update(h16_ref, hloc_ref, adj_ref, deg_ref, ce_ref,
                          wi_ref, wj_ref, we_ref)
    na = na_ref[...]
    t = (jnp.dot(_bf16(h_new * na), w1_ref[...],
                 preferred_element_type=jnp.float32) + b1_ref[...])
    t = t * jax.nn.sigmoid(t)
    o_ref[...] = (
        jnp.dot(_bf16(t * na), w2_ref[...],
                preferred_element_type=jnp.float32) + b2_ref[...]
    )


def _forward(ndev, n, x, na, dst, src, ea, v, ew, eb, lwi, lwj, lwe,
             w1, b1, w2, b2):
    blk = n // ndev
    nchunks, _, ec = dst.shape

    base = (jax.lax.axis_index("c") * blk).astype(jnp.int32) if ndev > 1 \
        else jnp.int32(0)
    dst_r = dst - base
    src_r, ea_r, v_r = src, ea, v

    adj16, small = pl.pallas_call(
        functools.partial(_build_kernel, blk=blk, n=n, nchunks=nchunks),
        out_shape=(jax.ShapeDtypeStruct((blk, n), jnp.bfloat16),
                   jax.ShapeDtypeStruct((blk, _LANE), jnp.float32)),
        grid=(nchunks,),
        in_specs=[
            pl.BlockSpec((1, 1, ec), lambda k: (k, 0, 0)),
            pl.BlockSpec((1, ec, 1), lambda k: (k, 0, 0)),
            pl.BlockSpec((1, ec, 1), lambda k: (k, 0, 0)),
            pl.BlockSpec((1, ec, _LANE), lambda k: (k, 0, 0)),
        ],
        out_specs=(pl.BlockSpec((blk, n), lambda k: (0, 0)),
                   pl.BlockSpec((blk, _LANE), lambda k: (0, 0))),
        scratch_shapes=[pltpu.VMEM((blk, n), jnp.float32),
                        pltpu.VMEM((blk, _LANE), jnp.float32)],
        compiler_params=pltpu.CompilerParams(
            dimension_semantics=("arbitrary",)),
    )(dst_r, src_r, ea_r, v_r)

    deg = small[:, _LANE - 1:]
    ce16 = _bf16(small)

    h = _call(_embed_kernel, (blk, ew.shape[1]), (x, na, ew, eb))
    for l in range(6):
        h16 = _bf16(h)
        if ndev > 1:
            h16 = jax.lax.all_gather(h16, "c", axis=0, tiled=True)
        ops = (h16, h, adj16, deg, ce16, lwi[l], lwj[l], lwe[l])
        if l < 5:
            h = _call(_layer_kernel, h.shape, ops)
        else:
            out = _call(_last_kernel, (blk, w2.shape[1]),
                        ops + (na, w1, b1, w2, b2))
    return out


@jax.jit
def kernel(x, edge_index, amf, node_attr, edge_attr, embed_w, embed_b,
           out1_w, out1_b, out2_w, out2_b, layer0_w, layer0_b, layer1_w,
           layer1_b, layer2_w, layer2_b, layer3_w, layer3_b, layer4_w,
           layer4_b, layer5_w, layer5_b):
    n, in_dim = x.shape
    e = edge_index.shape[1]
    hidden = embed_w.shape[1]
    a_dim = amf.shape[1]

    src, dst = edge_index[0], edge_index[1]
    ea = edge_attr[:, 0]

    eam = edge_attr * amf
    v = jnp.concatenate(
        [eam, jnp.ones((e, 1), jnp.float32),
         jnp.zeros((e, _LANE - a_dim - 2), jnp.float32),
         edge_attr], axis=1)

    lwi, lwj, lwe = [], [], []
    for w, b in ((layer0_w, layer0_b), (layer1_w, layer1_b),
                 (layer2_w, layer2_b), (layer3_w, layer3_b),
                 (layer4_w, layer4_b), (layer5_w, layer5_b)):
        lwi.append(_bf16(w[:hidden]))
        lwj.append(_bf16(w[hidden:2 * hidden]))
        we = jnp.concatenate(
            [w[2 * hidden:2 * hidden + a_dim], b,
             jnp.zeros((_LANE - a_dim - 1, hidden), jnp.float32)], axis=0)
        lwe.append(_bf16(we))
    lwi, lwj, lwe = jnp.stack(lwi), jnp.stack(lwj), jnp.stack(lwe)

    devs = jax.devices()
    ndev = 2 if (len(devs) >= 2 and n % 2 == 0) else 1
    mesh = Mesh(np.array(devs[:ndev]), ("c",))
    row = P("c", None)
    rep1 = P(None)
    rep2 = P(None, None)
    rep3 = P(None, None, None)

    ec = min(_ECHUNK, e)
    nchunks = e // ec
    out = jax.shard_map(
        functools.partial(_forward, ndev, n),
        mesh=mesh,
        in_specs=(row, row,
                  rep3, rep3, rep3, rep3,
                  rep2, rep2, rep3, rep3, rep3, rep2, rep2, rep2, rep2),
        out_specs=row,
        check_vma=False,
    )(x, node_attr,
      dst.reshape(nchunks, 1, ec), src.reshape(nchunks, ec, 1),
      ea.reshape(nchunks, ec, 1), _bf16(v).reshape(nchunks, ec, _LANE),
      _bf16(embed_w), embed_b, lwi, lwj, lwe,
      _bf16(out1_w), out1_b, _bf16(out2_w), out2_b)

    return out


# speedup vs baseline: 5.7966x; 2.9832x over previous
import functools

import jax
import jax.numpy as jnp
from jax.experimental import pallas as pl
from jax.experimental.pallas import tpu as pltpu

_LANE = 128
_ECHUNK = 2048


def _bf16(a):
    return a.astype(jnp.bfloat16)


def _dot(a, b):
    return jnp.dot(a, b, preferred_element_type=jnp.float32)


def _fused_kernel(dst_ref, src_ref, ea_ref, v_ref, x_ref, na_ref,
                  ew_ref, eb_ref, lwi_ref, lwj_ref, lwe_ref,
                  w1_ref, b1_ref, w2_ref, b2_ref,
                  o_ref, accf, sacc, adj16, hbuf, *, nchunks, n):
    k = pl.program_id(0)
    ec = dst_ref.shape[2]

    @pl.when(k == 0)
    def _():
        accf[...] = jnp.zeros_like(accf)
        sacc[...] = jnp.zeros_like(sacc)

    dstv = dst_ref[0]
    srcv = src_ref[0]
    eav = _bf16(ea_ref[0])
    ut = _bf16(jax.lax.broadcasted_iota(jnp.int32, (n, ec), 0) == dstv)
    sp = _bf16(jax.lax.broadcasted_iota(jnp.int32, (ec, n), 1) == srcv) * eav
    accf[...] += _dot(ut, sp)
    sacc[...] += _dot(ut, v_ref[0])

    @pl.when(k == nchunks - 1)
    def _():
        adj16[...] = _bf16(accf[...])
        deg = sacc[:, _LANE - 1:]
        ce16 = _bf16(sacc[...])
        na = na_ref[...]

        hbuf[...] = _dot(_bf16(x_ref[...] * na), ew_ref[...]) + eb_ref[...]

        def _layer(l, carry):
            h = hbuf[...]
            h16 = _bf16(h)
            ah = _dot(adj16[...], h16)
            agg = (
                deg * _dot(h16, lwi_ref[l])
                + _dot(_bf16(ah), lwj_ref[l])
                + _dot(ce16, lwe_ref[l])
            )
            hbuf[...] = h + agg * jax.nn.sigmoid(agg)
            return carry

        jax.lax.fori_loop(0, lwi_ref.shape[0], _layer, 0)

        h = hbuf[...]
        t = _dot(_bf16(h * na), w1_ref[...]) + b1_ref[...]
        t = t * jax.nn.sigmoid(t)
        o_ref[...] = _dot(_bf16(t * na), w2_ref[...]) + b2_ref[...]


@jax.jit
def kernel(x, edge_index, amf, node_attr, edge_attr, embed_w, embed_b,
           out1_w, out1_b, out2_w, out2_b, layer0_w, layer0_b, layer1_w,
           layer1_b, layer2_w, layer2_b, layer3_w, layer3_b, layer4_w,
           layer4_b, layer5_w, layer5_b):
    n, in_dim = x.shape
    e = edge_index.shape[1]
    hidden = embed_w.shape[1]
    out_dim = out2_w.shape[1]
    a_dim = amf.shape[1]
    ec = min(_ECHUNK, e)
    nchunks = e // ec

    src, dst = edge_index[0], edge_index[1]

    v = jnp.concatenate(
        [edge_attr * amf, jnp.ones((e, 1), jnp.float32),
         jnp.zeros((e, _LANE - a_dim - 2), jnp.float32),
         edge_attr], axis=1)

    lwi, lwj, lwe = [], [], []
    for w, b in ((layer0_w, layer0_b), (layer1_w, layer1_b),
                 (layer2_w, layer2_b), (layer3_w, layer3_b),
                 (layer4_w, layer4_b), (layer5_w, layer5_b)):
        lwi.append(_bf16(w[:hidden]))
        lwj.append(_bf16(w[hidden:2 * hidden]))
        we = jnp.concatenate(
            [w[2 * hidden:2 * hidden + a_dim], b,
             jnp.zeros((_LANE - a_dim - 1, hidden), jnp.float32)], axis=0)
        lwe.append(_bf16(we))
    lwi, lwj, lwe = jnp.stack(lwi), jnp.stack(lwj), jnp.stack(lwe)

    def _chunk3(kk):
        return lambda k: (k,) + (0,) * (kk - 1)

    def _const(shape):
        zeros = (0,) * len(shape)
        return pl.BlockSpec(shape, lambda k, _z=zeros: _z)

    nl = lwi.shape[0]
    out = pl.pallas_call(
        functools.partial(_fused_kernel, nchunks=nchunks, n=n),
        out_shape=jax.ShapeDtypeStruct((n, out_dim), jnp.float32),
        grid=(nchunks,),
        in_specs=[
            pl.BlockSpec((1, 1, ec), _chunk3(3)),
            pl.BlockSpec((1, ec, 1), _chunk3(3)),
            pl.BlockSpec((1, ec, 1), _chunk3(3)),
            pl.BlockSpec((1, ec, _LANE), _chunk3(3)),
            _const((n, in_dim)),
            _const((n, 1)),
            _const((in_dim, hidden)),
            _const((1, hidden)),
            _const((nl, hidden, hidden)),
            _const((nl, hidden, hidden)),
            _const((nl, _LANE, hidden)),
            _const((hidden, hidden)),
            _const((1, hidden)),
            _const((hidden, out_dim)),
            _const((1, out_dim)),
        ],
        out_specs=pl.BlockSpec((n, out_dim), lambda k: (0, 0)),
        scratch_shapes=[
            pltpu.VMEM((n, n), jnp.float32),
            pltpu.VMEM((n, _LANE), jnp.float32),
            pltpu.VMEM((n, n), jnp.bfloat16),
            pltpu.VMEM((n, hidden), jnp.float32),
        ],
        compiler_params=pltpu.CompilerParams(
            dimension_semantics=("arbitrary",),
            vmem_limit_bytes=56 << 20),
    )(dst.reshape(nchunks, 1, ec), src.reshape(nchunks, ec, 1),
      edge_attr[:, 0].reshape(nchunks, ec, 1),
      _bf16(v).reshape(nchunks, ec, _LANE),
      x, node_attr, _bf16(embed_w), embed_b, lwi, lwj, lwe,
      _bf16(out1_w), out1_b, _bf16(out2_w), out2_b)

    return out


# speedup vs baseline: 8.6097x; 1.4853x over previous
import functools

import jax
import jax.numpy as jnp
from jax.experimental import pallas as pl
from jax.experimental.pallas import tpu as pltpu

_ECHUNK = 2048


def _bf16(a):
    return a.astype(jnp.bfloat16)


def _dot(a, b):
    return jnp.dot(a, b, preferred_element_type=jnp.float32)


def _dot_tb(a, b):
    return jax.lax.dot_general(a, b, (((1,), (1,)), ((), ())),
                               preferred_element_type=jnp.float32)


def _fused_kernel(ei_ref, ea_ref, amf_ref, x_ref, na_ref, ew_ref, eb_ref,
                  w0_ref, b0_ref, w1_ref, b1_ref, w2_ref, b2_ref,
                  w3_ref, b3_ref, w4_ref, b4_ref, w5_ref, b5_ref,
                  o1w_ref, o1b_ref, o2w_ref, o2b_ref,
                  o_ref, accf, sacc, adj16, hbuf,
                  *, nchunks, n, hidden, a_dim):
    k = pl.program_id(0)
    ec = ei_ref.shape[1]

    @pl.when(k == 0)
    def _():
        accf[...] = jnp.zeros_like(accf)
        sacc[...] = jnp.zeros_like(sacc)

    srcv = ei_ref[0:1, :]
    dstv = ei_ref[1:2, :]
    eav = _bf16(ea_ref[...])
    ut = _bf16(jax.lax.broadcasted_iota(jnp.int32, (n, ec), 0) == dstv)
    st = _bf16(jax.lax.broadcasted_iota(jnp.int32, (n, ec), 0) == srcv)
    accf[...] += _dot_tb(ut * eav.reshape(1, ec), st)
    vals = jnp.concatenate(
        [_bf16(amf_ref[...]) * eav, jnp.ones((ec, 1), jnp.bfloat16), eav],
        axis=1)
    sacc[...] += _dot(ut, vals)

    @pl.when(k == nchunks - 1)
    def _():
        adj16[...] = _bf16(accf[...])
        ce16 = _bf16(sacc[:, :a_dim])
        cnt = sacc[:, a_dim:a_dim + 1]
        deg = sacc[:, a_dim + 1:a_dim + 2]
        na = na_ref[...]

        hbuf[...] = (_dot(_bf16(x_ref[...] * na), _bf16(ew_ref[...]))
                     + eb_ref[...])

        for w_ref, b_ref in ((w0_ref, b0_ref), (w1_ref, b1_ref),
                             (w2_ref, b2_ref), (w3_ref, b3_ref),
                             (w4_ref, b4_ref), (w5_ref, b5_ref)):
            h = hbuf[...]
            h16 = _bf16(h)
            ah = _dot(adj16[...], h16)
            agg = (
                deg * _dot(h16, _bf16(w_ref[:hidden]))
                + _dot(_bf16(ah), _bf16(w_ref[hidden:2 * hidden]))
                + _dot(ce16, _bf16(w_ref[2 * hidden:2 * hidden + a_dim]))
                + cnt * b_ref[...]
            )
            hbuf[...] = h + agg * jax.nn.sigmoid(agg)

        h = hbuf[...]
        t = _dot(_bf16(h * na), _bf16(o1w_ref[...])) + o1b_ref[...]
        t = t * jax.nn.sigmoid(t)
        o_ref[...] = _dot(_bf16(t * na), _bf16(o2w_ref[...])) + o2b_ref[...]


@jax.jit
def kernel(x, edge_index, amf, node_attr, edge_attr, embed_w, embed_b,
           out1_w, out1_b, out2_w, out2_b, layer0_w, layer0_b, layer1_w,
           layer1_b, layer2_w, layer2_b, layer3_w, layer3_b, layer4_w,
           layer4_b, layer5_w, layer5_b):
    n, in_dim = x.shape
    e = edge_index.shape[1]
    hidden = embed_w.shape[1]
    out_dim = out2_w.shape[1]
    a_dim = amf.shape[1]
    wrows = layer0_w.shape[0]
    ec = min(_ECHUNK, e)
    nchunks = e // ec

    def _const(shape):
        zeros = (0,) * len(shape)
        return pl.BlockSpec(shape, lambda k, _z=zeros: _z)

    lspecs = []
    for _ in range(6):
        lspecs += [_const((wrows, hidden)), _const((1, hidden))]

    out = pl.pallas_call(
        functools.partial(_fused_kernel, nchunks=nchunks, n=n,
                          hidden=hidden, a_dim=a_dim),
        out_shape=jax.ShapeDtypeStruct((n, out_dim), jnp.float32),
        grid=(nchunks,),
        in_specs=[
            pl.BlockSpec((2, ec), lambda k: (0, k)),
            pl.BlockSpec((ec, 1), lambda k: (k, 0)),
            pl.BlockSpec((ec, a_dim), lambda k: (k, 0)),
            _const((n, in_dim)),
            _const((n, 1)),
            _const((in_dim, hidden)),
            _const((1, hidden)),
        ] + lspecs + [
            _const((hidden, hidden)),
            _const((1, hidden)),
            _const((hidden, out_dim)),
            _const((1, out_dim)),
        ],
        out_specs=pl.BlockSpec((n, out_dim), lambda k: (0, 0)),
        scratch_shapes=[
            pltpu.VMEM((n, n), jnp.float32),
            pltpu.VMEM((n, a_dim + 2), jnp.float32),
            pltpu.VMEM((n, n), jnp.bfloat16),
            pltpu.VMEM((n, hidden), jnp.float32),
        ],
        compiler_params=pltpu.CompilerParams(
            dimension_semantics=("arbitrary",),
            vmem_limit_bytes=56 << 20),
    )(edge_index, edge_attr, amf, x, node_attr, embed_w, embed_b,
      layer0_w, layer0_b, layer1_w, layer1_b, layer2_w, layer2_b,
      layer3_w, layer3_b, layer4_w, layer4_b, layer5_w, layer5_b,
      out1_w, out1_b, out2_w, out2_b)

    return out


# speedup vs baseline: 9.0196x; 1.0476x over previous
import functools

import jax
import jax.numpy as jnp
from jax.experimental import pallas as pl
from jax.experimental.pallas import tpu as pltpu

_ECHUNK = 2048


def _bf16(a):
    return a.astype(jnp.bfloat16)


def _dot(a, b):
    return jnp.dot(a, b, preferred_element_type=jnp.float32)


def _dot_tb(a, b):
    return jax.lax.dot_general(a, b, (((1,), (1,)), ((), ())),
                               preferred_element_type=jnp.float32)


def _fused_kernel(ei_ref, v_ref, x_ref, na_ref, ew_ref, eb_ref,
                  w0_ref, b0_ref, w1_ref, b1_ref, w2_ref, b2_ref,
                  w3_ref, b3_ref, w4_ref, b4_ref, w5_ref, b5_ref,
                  o1w_ref, o1b_ref, o2w_ref, o2b_ref,
                  o_ref, accf, sacc, adj16, hbuf,
                  *, nchunks, n, hidden, a_dim):
    k = pl.program_id(0)
    ec = ei_ref.shape[1]

    @pl.when(k == 0)
    def _():
        accf[...] = jnp.zeros_like(accf)
        sacc[...] = jnp.zeros_like(sacc)

    srcv = ei_ref[0:1, :]
    dstv = ei_ref[1:2, :]
    vals = _bf16(v_ref[...])
    eav = vals[:, a_dim + 1:]
    ut = _bf16(jax.lax.broadcasted_iota(jnp.int32, (n, ec), 0) == dstv)
    st = _bf16(jax.lax.broadcasted_iota(jnp.int32, (n, ec), 0) == srcv)
    accf[...] += _dot_tb(ut * eav.reshape(1, ec), st)
    sacc[...] += _dot(ut, vals)

    @pl.when(k == nchunks - 1)
    def _():
        adj16[...] = _bf16(accf[...])
        ce16 = _bf16(sacc[:, :a_dim])
        cnt = sacc[:, a_dim:a_dim + 1]
        deg = sacc[:, a_dim + 1:a_dim + 2]
        na = na_ref[...]

        hbuf[...] = (_dot(_bf16(x_ref[...] * na), _bf16(ew_ref[...]))
                     + eb_ref[...])

        for w_ref, b_ref in ((w0_ref, b0_ref), (w1_ref, b1_ref),
                             (w2_ref, b2_ref), (w3_ref, b3_ref),
                             (w4_ref, b4_ref), (w5_ref, b5_ref)):
            h = hbuf[...]
            h16 = _bf16(h)
            ah = _dot(adj16[...], h16)
            agg = (
                deg * _dot(h16, _bf16(w_ref[:hidden]))
                + _dot(_bf16(ah), _bf16(w_ref[hidden:2 * hidden]))
                + _dot(ce16, _bf16(w_ref[2 * hidden:2 * hidden + a_dim]))
                + cnt * b_ref[...]
            )
            hbuf[...] = h + agg * jax.nn.sigmoid(agg)

        h = hbuf[...]
        t = _dot(_bf16(h * na), _bf16(o1w_ref[...])) + o1b_ref[...]
        t = t * jax.nn.sigmoid(t)
        o_ref[...] = _dot(_bf16(t * na), _bf16(o2w_ref[...])) + o2b_ref[...]


@jax.jit
def kernel(x, edge_index, amf, node_attr, edge_attr, embed_w, embed_b,
           out1_w, out1_b, out2_w, out2_b, layer0_w, layer0_b, layer1_w,
           layer1_b, layer2_w, layer2_b, layer3_w, layer3_b, layer4_w,
           layer4_b, layer5_w, layer5_b):
    n, in_dim = x.shape
    e = edge_index.shape[1]
    hidden = embed_w.shape[1]
    out_dim = out2_w.shape[1]
    a_dim = amf.shape[1]
    wrows = layer0_w.shape[0]
    ec = min(_ECHUNK, e)
    nchunks = e // ec

    v = jnp.concatenate(
        [edge_attr * amf, jnp.ones((e, 1), jnp.float32), edge_attr], axis=1)

    def _const(shape):
        zeros = (0,) * len(shape)
        return pl.BlockSpec(shape, lambda k, _z=zeros: _z)

    lspecs = []
    for _ in range(6):
        lspecs += [_const((wrows, hidden)), _const((1, hidden))]

    out = pl.pallas_call(
        functools.partial(_fused_kernel, nchunks=nchunks, n=n,
                          hidden=hidden, a_dim=a_dim),
        out_shape=jax.ShapeDtypeStruct((n, out_dim), jnp.float32),
        grid=(nchunks,),
        in_specs=[
            pl.BlockSpec((2, ec), lambda k: (0, k)),
            pl.BlockSpec((ec, a_dim + 2), lambda k: (k, 0)),
            _const((n, in_dim)),
            _const((n, 1)),
            _const((in_dim, hidden)),
            _const((1, hidden)),
        ] + lspecs + [
            _const((hidden, hidden)),
            _const((1, hidden)),
            _const((hidden, out_dim)),
            _const((1, out_dim)),
        ],
        out_specs=pl.BlockSpec((n, out_dim), lambda k: (0, 0)),
        scratch_shapes=[
            pltpu.VMEM((n, n), jnp.float32),
            pltpu.VMEM((n, a_dim + 2), jnp.float32),
            pltpu.VMEM((n, n), jnp.bfloat16),
            pltpu.VMEM((n, hidden), jnp.float32),
        ],
        compiler_params=pltpu.CompilerParams(
            dimension_semantics=("arbitrary",),
            vmem_limit_bytes=56 << 20),
    )(edge_index, v, x, node_attr, embed_w, embed_b,
      layer0_w, layer0_b, layer1_w, layer1_b, layer2_w, layer2_b,
      layer3_w, layer3_b, layer4_w, layer4_b, layer5_w, layer5_b,
      out1_w, out1_b, out2_w, out2_b)

    return out


# speedup vs baseline: 9.8321x; 1.0901x over previous
import functools

import jax
import jax.numpy as jnp
from jax.experimental import pallas as pl
from jax.experimental.pallas import tpu as pltpu

_ECHUNK = 2048


def _bf16(a):
    return a.astype(jnp.bfloat16)


def _dot(a, b):
    return jnp.dot(a, b, preferred_element_type=jnp.float32)


def _dot_tb(a, b):
    return jax.lax.dot_general(a, b, (((1,), (1,)), ((), ())),
                               preferred_element_type=jnp.float32)


def _fused_kernel(ei_ref, v_ref, x_ref, na_ref, ew_ref, eb_ref,
                  w0_ref, b0_ref, w1_ref, b1_ref, w2_ref, b2_ref,
                  w3_ref, b3_ref, w4_ref, b4_ref, w5_ref, b5_ref,
                  o1w_ref, o1b_ref, o2w_ref, o2b_ref,
                  o_ref, accf, sacc, adj16, hbuf,
                  *, nchunks, n, hidden, a_dim):
    k = pl.program_id(0)
    ec = ei_ref.shape[1]

    @pl.when(k == 0)
    def _():
        accf[...] = jnp.zeros_like(accf)
        sacc[...] = jnp.zeros_like(sacc)

    srcv = ei_ref[0:1, :]
    dstv = ei_ref[1:2, :]
    vals = _bf16(v_ref[...])
    eav = vals[a_dim + 1:, :]
    ut = _bf16(jax.lax.broadcasted_iota(jnp.int32, (n, ec), 0) == dstv)
    st = _bf16(jax.lax.broadcasted_iota(jnp.int32, (n, ec), 0) == srcv)
    accf[...] += _dot_tb(ut * eav, st)
    sacc[...] += _dot_tb(ut, vals)

    @pl.when(k == nchunks - 1)
    def _():
        adj16[...] = _bf16(accf[...])
        ce16 = _bf16(sacc[:, :a_dim])
        cnt = sacc[:, a_dim:a_dim + 1]
        deg = sacc[:, a_dim + 1:a_dim + 2]
        na = na_ref[...].reshape(n, 1)

        hbuf[...] = (_dot(_bf16(x_ref[...] * na), _bf16(ew_ref[...]))
                     + eb_ref[...])

        for w_ref, b_ref in ((w0_ref, b0_ref), (w1_ref, b1_ref),
                             (w2_ref, b2_ref), (w3_ref, b3_ref),
                             (w4_ref, b4_ref), (w5_ref, b5_ref)):
            h = hbuf[...]
            h16 = _bf16(h)
            ah = _dot(adj16[...], h16)
            agg = (
                deg * _dot(h16, _bf16(w_ref[:hidden]))
                + _dot(_bf16(ah), _bf16(w_ref[hidden:2 * hidden]))
                + _dot(ce16, _bf16(w_ref[2 * hidden:2 * hidden + a_dim]))
                + cnt * b_ref[...]
            )
            hbuf[...] = h + agg * jax.nn.sigmoid(agg)

        h = hbuf[...]
        t = _dot(_bf16(h * na), _bf16(o1w_ref[...])) + o1b_ref[...]
        t = t * jax.nn.sigmoid(t)
        o_ref[...] = _dot(_bf16(t * na), _bf16(o2w_ref[...])) + o2b_ref[...]


@jax.jit
def kernel(x, edge_index, amf, node_attr, edge_attr, embed_w, embed_b,
           out1_w, out1_b, out2_w, out2_b, layer0_w, layer0_b, layer1_w,
           layer1_b, layer2_w, layer2_b, layer3_w, layer3_b, layer4_w,
           layer4_b, layer5_w, layer5_b):
    n, in_dim = x.shape
    e = edge_index.shape[1]
    hidden = embed_w.shape[1]
    out_dim = out2_w.shape[1]
    a_dim = amf.shape[1]
    wrows = layer0_w.shape[0]
    ec = min(_ECHUNK, e)
    nchunks = e // ec

    ea_row = edge_attr.T
    v = jnp.concatenate(
        [amf.T * ea_row, jnp.ones((1, e), jnp.float32), ea_row], axis=0)

    def _const(shape):
        zeros = (0,) * len(shape)
        return pl.BlockSpec(shape, lambda k, _z=zeros: _z)

    lspecs = []
    for _ in range(6):
        lspecs += [_const((wrows, hidden)), _const((1, hidden))]

    out = pl.pallas_call(
        functools.partial(_fused_kernel, nchunks=nchunks, n=n,
                          hidden=hidden, a_dim=a_dim),
        out_shape=jax.ShapeDtypeStruct((n, out_dim), jnp.float32),
        grid=(nchunks,),
        in_specs=[
            pl.BlockSpec((2, ec), lambda k: (0, k)),
            pl.BlockSpec((a_dim + 2, ec), lambda k: (0, k)),
            _const((n, in_dim)),
            _const((1, n)),
            _const((in_dim, hidden)),
            _const((1, hidden)),
        ] + lspecs + [
            _const((hidden, hidden)),
            _const((1, hidden)),
            _const((hidden, out_dim)),
            _const((1, out_dim)),
        ],
        out_specs=pl.BlockSpec((n, out_dim), lambda k: (0, 0)),
        scratch_shapes=[
            pltpu.VMEM((n, n), jnp.float32),
            pltpu.VMEM((n, a_dim + 2), jnp.float32),
            pltpu.VMEM((n, n), jnp.bfloat16),
            pltpu.VMEM((n, hidden), jnp.float32),
        ],
        compiler_params=pltpu.CompilerParams(
            dimension_semantics=("arbitrary",),
            vmem_limit_bytes=56 << 20),
    )(edge_index, v, x, node_attr.T, embed_w, embed_b,
      layer0_w, layer0_b, layer1_w, layer1_b, layer2_w, layer2_b,
      layer3_w, layer3_b, layer4_w, layer4_b, layer5_w, layer5_b,
      out1_w, out1_b, out2_w, out2_b)

    return out


# speedup vs baseline: 9.9876x; 1.0158x over previous
import functools

import jax
import jax.numpy as jnp
from jax.experimental import pallas as pl
from jax.experimental.pallas import tpu as pltpu

_ECHUNK = 4096


def _bf16(a):
    return a.astype(jnp.bfloat16)


def _dot(a, b):
    return jnp.dot(a, b, preferred_element_type=jnp.float32)


def _dot_tb(a, b):
    return jax.lax.dot_general(a, b, (((1,), (1,)), ((), ())),
                               preferred_element_type=jnp.float32)


def _fused_kernel(ei_ref, v_ref, x_ref, na_ref, ew_ref, eb_ref,
                  w0_ref, b0_ref, w1_ref, b1_ref, w2_ref, b2_ref,
                  w3_ref, b3_ref, w4_ref, b4_ref, w5_ref, b5_ref,
                  o1w_ref, o1b_ref, o2w_ref, o2b_ref,
                  o_ref, accf, sacc, adj16, hbuf,
                  *, nchunks, n, hidden, a_dim):
    k = pl.program_id(0)
    ec = ei_ref.shape[1]

    @pl.when(k == 0)
    def _():
        accf[...] = jnp.zeros_like(accf)
        sacc[...] = jnp.zeros_like(sacc)

    srcv = ei_ref[0:1, :]
    dstv = ei_ref[1:2, :]
    vals = _bf16(v_ref[...])
    eav = vals[a_dim + 1:, :]
    ut = _bf16(jax.lax.broadcasted_iota(jnp.int32, (n, ec), 0) == dstv)
    st = _bf16(jax.lax.broadcasted_iota(jnp.int32, (n, ec), 0) == srcv)
    accf[...] += _dot_tb(ut * eav, st)
    sacc[...] += _dot_tb(ut, vals)

    @pl.when(k == nchunks - 1)
    def _():
        adj16[...] = _bf16(accf[...])
        ce16 = _bf16(sacc[:, :a_dim])
        cnt = sacc[:, a_dim:a_dim + 1]
        deg = sacc[:, a_dim + 1:a_dim + 2]
        na = na_ref[...].reshape(n, 1)

        hbuf[...] = (_dot(_bf16(x_ref[...] * na), _bf16(ew_ref[...]))
                     + eb_ref[...])

        for w_ref, b_ref in ((w0_ref, b0_ref), (w1_ref, b1_ref),
                             (w2_ref, b2_ref), (w3_ref, b3_ref),
                             (w4_ref, b4_ref), (w5_ref, b5_ref)):
            h = hbuf[...]
            h16 = _bf16(h)
            ah = _dot(adj16[...], h16)
            agg = (
                deg * _dot(h16, _bf16(w_ref[:hidden]))
                + _dot(_bf16(ah), _bf16(w_ref[hidden:2 * hidden]))
                + _dot(ce16, _bf16(w_ref[2 * hidden:2 * hidden + a_dim]))
                + cnt * b_ref[...]
            )
            hbuf[...] = h + agg * jax.nn.sigmoid(agg)

        h = hbuf[...]
        t = _dot(_bf16(h * na), _bf16(o1w_ref[...])) + o1b_ref[...]
        t = t * jax.nn.sigmoid(t)
        o_ref[...] = _dot(_bf16(t * na), _bf16(o2w_ref[...])) + o2b_ref[...]


@jax.jit
def kernel(x, edge_index, amf, node_attr, edge_attr, embed_w, embed_b,
           out1_w, out1_b, out2_w, out2_b, layer0_w, layer0_b, layer1_w,
           layer1_b, layer2_w, layer2_b, layer3_w, layer3_b, layer4_w,
           layer4_b, layer5_w, layer5_b):
    n, in_dim = x.shape
    e = edge_index.shape[1]
    hidden = embed_w.shape[1]
    out_dim = out2_w.shape[1]
    a_dim = amf.shape[1]
    wrows = layer0_w.shape[0]
    ec = min(_ECHUNK, e)
    nchunks = e // ec

    ea_row = edge_attr.T
    v = jnp.concatenate(
        [amf.T * ea_row, jnp.ones((1, e), jnp.float32), ea_row], axis=0)

    def _const(shape):
        zeros = (0,) * len(shape)
        return pl.BlockSpec(shape, lambda k, _z=zeros: _z)

    lspecs = []
    for _ in range(6):
        lspecs += [_const((wrows, hidden)), _const((1, hidden))]

    out = pl.pallas_call(
        functools.partial(_fused_kernel, nchunks=nchunks, n=n,
                          hidden=hidden, a_dim=a_dim),
        out_shape=jax.ShapeDtypeStruct((n, out_dim), jnp.float32),
        grid=(nchunks,),
        in_specs=[
            pl.BlockSpec((2, ec), lambda k: (0, k)),
            pl.BlockSpec((a_dim + 2, ec), lambda k: (0, k)),
            _const((n, in_dim)),
            _const((1, n)),
            _const((in_dim, hidden)),
            _const((1, hidden)),
        ] + lspecs + [
            _const((hidden, hidden)),
            _const((1, hidden)),
            _const((hidden, out_dim)),
            _const((1, out_dim)),
        ],
        out_specs=pl.BlockSpec((n, out_dim), lambda k: (0, 0)),
        scratch_shapes=[
            pltpu.VMEM((n, n), jnp.float32),
            pltpu.VMEM((n, a_dim + 2), jnp.float32),
            pltpu.VMEM((n, n), jnp.bfloat16),
            pltpu.VMEM((n, hidden), jnp.float32),
        ],
        compiler_params=pltpu.CompilerParams(
            dimension_semantics=("arbitrary",),
            vmem_limit_bytes=56 << 20),
    )(edge_index, v, x, node_attr.T, embed_w, embed_b,
      layer0_w, layer0_b, layer1_w, layer1_b, layer2_w, layer2_b,
      layer3_w, layer3_b, layer4_w, layer4_b, layer5_w, layer5_b,
      out1_w, out1_b, out2_w, out2_b)

    return out


# speedup vs baseline: 10.0782x; 1.0091x over previous
import functools

import jax
import jax.numpy as jnp
from jax.experimental import pallas as pl
from jax.experimental.pallas import tpu as pltpu

_ECHUNK = 8192


def _bf16(a):
    return a.astype(jnp.bfloat16)


def _dot(a, b):
    return jnp.dot(a, b, preferred_element_type=jnp.float32)


def _dot_tb(a, b):
    return jax.lax.dot_general(a, b, (((1,), (1,)), ((), ())),
                               preferred_element_type=jnp.float32)


def _fused_kernel(ei_ref, v_ref, x_ref, na_ref, ew_ref, eb_ref,
                  w0_ref, b0_ref, w1_ref, b1_ref, w2_ref, b2_ref,
                  w3_ref, b3_ref, w4_ref, b4_ref, w5_ref, b5_ref,
                  o1w_ref, o1b_ref, o2w_ref, o2b_ref,
                  o_ref, accf, sacc, adj16, hbuf,
                  *, nchunks, n, hidden, a_dim):
    k = pl.program_id(0)
    ec = ei_ref.shape[1]

    @pl.when(k == 0)
    def _():
        accf[...] = jnp.zeros_like(accf)
        sacc[...] = jnp.zeros_like(sacc)

    srcv = ei_ref[0:1, :]
    dstv = ei_ref[1:2, :]
    vals = _bf16(v_ref[...])
    eav = vals[a_dim + 1:, :]
    ut = _bf16(jax.lax.broadcasted_iota(jnp.int32, (n, ec), 0) == dstv)
    st = _bf16(jax.lax.broadcasted_iota(jnp.int32, (n, ec), 0) == srcv)
    accf[...] += _dot_tb(ut * eav, st)
    sacc[...] += _dot_tb(ut, vals)

    @pl.when(k == nchunks - 1)
    def _():
        adj16[...] = _bf16(accf[...])
        ce16 = _bf16(sacc[:, :a_dim])
        cnt = sacc[:, a_dim:a_dim + 1]
        deg = sacc[:, a_dim + 1:a_dim + 2]
        na = na_ref[...].reshape(n, 1)

        hbuf[...] = (_dot(_bf16(x_ref[...] * na), _bf16(ew_ref[...]))
                     + eb_ref[...])

        for w_ref, b_ref in ((w0_ref, b0_ref), (w1_ref, b1_ref),
                             (w2_ref, b2_ref), (w3_ref, b3_ref),
                             (w4_ref, b4_ref), (w5_ref, b5_ref)):
            h = hbuf[...]
            h16 = _bf16(h)
            ah = _dot(adj16[...], h16)
            agg = (
                deg * _dot(h16, _bf16(w_ref[:hidden]))
                + _dot(_bf16(ah), _bf16(w_ref[hidden:2 * hidden]))
                + _dot(ce16, _bf16(w_ref[2 * hidden:2 * hidden + a_dim]))
                + cnt * b_ref[...]
            )
            hbuf[...] = h + agg * jax.nn.sigmoid(agg)

        h = hbuf[...]
        t = _dot(_bf16(h * na), _bf16(o1w_ref[...])) + o1b_ref[...]
        t = t * jax.nn.sigmoid(t)
        o_ref[...] = _dot(_bf16(t * na), _bf16(o2w_ref[...])) + o2b_ref[...]


@jax.jit
def kernel(x, edge_index, amf, node_attr, edge_attr, embed_w, embed_b,
           out1_w, out1_b, out2_w, out2_b, layer0_w, layer0_b, layer1_w,
           layer1_b, layer2_w, layer2_b, layer3_w, layer3_b, layer4_w,
           layer4_b, layer5_w, layer5_b):
    n, in_dim = x.shape
    e = edge_index.shape[1]
    hidden = embed_w.shape[1]
    out_dim = out2_w.shape[1]
    a_dim = amf.shape[1]
    wrows = layer0_w.shape[0]
    ec = min(_ECHUNK, e)
    nchunks = e // ec

    ea_row = edge_attr.T
    v = jnp.concatenate(
        [amf.T * ea_row, jnp.ones((1, e), jnp.float32), ea_row], axis=0)

    def _const(shape):
        zeros = (0,) * len(shape)
        return pl.BlockSpec(shape, lambda k, _z=zeros: _z)

    lspecs = []
    for _ in range(6):
        lspecs += [_const((wrows, hidden)), _const((1, hidden))]

    out = pl.pallas_call(
        functools.partial(_fused_kernel, nchunks=nchunks, n=n,
                          hidden=hidden, a_dim=a_dim),
        out_shape=jax.ShapeDtypeStruct((n, out_dim), jnp.float32),
        grid=(nchunks,),
        in_specs=[
            pl.BlockSpec((2, ec), lambda k: (0, k)),
            pl.BlockSpec((a_dim + 2, ec), lambda k: (0, k)),
            _const((n, in_dim)),
            _const((1, n)),
            _const((in_dim, hidden)),
            _const((1, hidden)),
        ] + lspecs + [
            _const((hidden, hidden)),
            _const((1, hidden)),
            _const((hidden, out_dim)),
            _const((1, out_dim)),
        ],
        out_specs=pl.BlockSpec((n, out_dim), lambda k: (0, 0)),
        scratch_shapes=[
            pltpu.VMEM((n, n), jnp.float32),
            pltpu.VMEM((n, a_dim + 2), jnp.float32),
            pltpu.VMEM((n, n), jnp.bfloat16),
            pltpu.VMEM((n, hidden), jnp.float32),
        ],
        compiler_params=pltpu.CompilerParams(
            dimension_semantics=("arbitrary",),
            vmem_limit_bytes=58 << 20),
    )(edge_index, v, x, node_attr.T, embed_w, embed_b,
      layer0_w, layer0_b, layer1_w, layer1_b, layer2_w, layer2_b,
      layer3_w, layer3_b, layer4_w, layer4_b, layer5_w, layer5_b,
      out1_w, out1_b, out2_w, out2_b)

    return out
